# baseline (speedup 1.0000x reference)
import functools

import jax
import jax.numpy as jnp
from jax import lax
from jax.experimental import pallas as pl
from jax.experimental.pallas import tpu as pltpu
from jax.experimental.pallas import tpu_sc as plsc

SP_LEN = 2048
EMBED_DIM = 4096

NUM_CORES = 2
NUM_SUBCORES = 16
NUM_WORKERS = NUM_CORES * NUM_SUBCORES
ROWS_PER_WORKER = SP_LEN // NUM_WORKERS


def _gather_body(table_hbm, idx_hbm, out_hbm, idx_v, sem):
    wid = lax.axis_index("s") * NUM_CORES + lax.axis_index("c")
    base = wid * ROWS_PER_WORKER

    pltpu.sync_copy(idx_hbm.at[wid], idx_v)
    pltpu.async_copy(
        table_hbm.at[pl.ds(base, ROWS_PER_WORKER)],
        out_hbm.at[pl.ds(base, ROWS_PER_WORKER)],
        sem,
    ).wait()


@jax.jit
def _soft_prompt_lookup(soft_prompt, seq_indices):
    idx = jnp.reshape(
        seq_indices.astype(jnp.int32), (NUM_WORKERS, ROWS_PER_WORKER)
    )
    mesh = plsc.VectorSubcoreMesh(core_axis_name="c", subcore_axis_name="s")
    run = functools.partial(
        pl.kernel,
        mesh=mesh,
        out_type=jax.ShapeDtypeStruct((SP_LEN, EMBED_DIM), jnp.float32),
        scratch_types=[
            pltpu.VMEM((ROWS_PER_WORKER,), jnp.int32),
            pltpu.SemaphoreType.DMA,
        ],
    )(_gather_body)
    return run(soft_prompt, idx)


def kernel(soft_prompt, seq_indices):
    return _soft_prompt_lookup(soft_prompt, seq_indices)


# speedup vs baseline: 24.2874x; 24.2874x over previous
import functools

import jax
import jax.numpy as jnp
from jax import lax
from jax.experimental import pallas as pl
from jax.experimental.pallas import tpu as pltpu
from jax.experimental.pallas import tpu_sc as plsc

SP_LEN = 2048
EMBED_DIM = 4096

NUM_CORES = 2
NUM_SUBCORES = 16
NUM_WORKERS = NUM_CORES * NUM_SUBCORES
ROWS_PER_WORKER = SP_LEN // NUM_WORKERS
CHUNK = 4
NUM_CHUNKS = ROWS_PER_WORKER // CHUNK
NBUF = 7


def _gather_body(table_hbm, idx_hbm, out_hbm, idx_v, rows_v, gsem, ssem):
    wid = lax.axis_index("s") * NUM_CORES + lax.axis_index("c")
    base = wid * ROWS_PER_WORKER

    pltpu.sync_copy(idx_hbm.at[wid], idx_v)

    gathers = [None] * NBUF
    stores = [None] * NBUF

    def start_gather(c):
        slot = c % NBUF
        gathers[slot] = pltpu.async_copy(
            table_hbm.at[idx_v.at[c]], rows_v.at[slot], gsem.at[slot]
        )

    for c in range(min(NBUF, NUM_CHUNKS)):
        start_gather(c)

    for c in range(NUM_CHUNKS):
        slot = c % NBUF
        gathers[slot].wait()
        stores[slot] = pltpu.async_copy(
            rows_v.at[slot],
            out_hbm.at[pl.ds(base + c * CHUNK, CHUNK)],
            ssem.at[slot],
        )
        nxt = c + NBUF
        if nxt < NUM_CHUNKS:
            stores[slot].wait()
            start_gather(nxt)

    for c in range(NUM_CHUNKS - NBUF, NUM_CHUNKS):
        if c >= 0:
            stores[c % NBUF].wait()


@jax.jit
def _soft_prompt_lookup(soft_prompt, seq_indices):
    idx = jnp.reshape(
        seq_indices.astype(jnp.int32), (NUM_WORKERS, NUM_CHUNKS, CHUNK)
    )
    mesh = plsc.VectorSubcoreMesh(core_axis_name="c", subcore_axis_name="s")
    run = functools.partial(
        pl.kernel,
        mesh=mesh,
        out_type=jax.ShapeDtypeStruct((SP_LEN, EMBED_DIM), jnp.float32),
        scratch_types=[
            pltpu.VMEM((NUM_CHUNKS, CHUNK), jnp.int32),
            pltpu.VMEM((NBUF, CHUNK, EMBED_DIM), jnp.float32),
            pltpu.SemaphoreType.DMA((NBUF,)),
            pltpu.SemaphoreType.DMA((NBUF,)),
        ],
    )(_gather_body)
    return run(soft_prompt, idx)


def kernel(soft_prompt, seq_indices):
    return _soft_prompt_lookup(soft_prompt, seq_indices)
